# Initial kernel scaffold; baseline (speedup 1.0000x reference)
import jax
import jax.numpy as jnp
from jax.experimental import pallas as pl


def kernel(x, x_lengths, tone, language, ja_bert, noise_scale, emb_w, tone_w, lang_w):
    raise NotImplementedError("write your pallas kernel here")


# speedup vs baseline: 1.0887x; 1.0887x over previous
import functools

import jax
import jax.numpy as jnp
from jax import lax
from jax.experimental import pallas as pl
from jax.experimental.pallas import tpu as pltpu
from jax.experimental.pallas import tpu_sc as plsc

N_VOCAB = 100000
NUM_TONES = 16
NUM_LANGUAGES = 10
HIDDEN = 192
B = 64
L = 512
N = B * L
LANES = 16
NSLICE = HIDDEN // LANES

NC = 2
NS = 16
NW = NC * NS
ROWS_PER_W = N // NW
CHUNK = 256
NCHUNK = ROWS_PER_W // CHUNK
B_PER_W = B // NW


def _sc_body(x_hbm, tone_hbm, lang_hbm, xlen_hbm, emb_hbm, tonew_hbm, langw_hbm,
             out_h, out_m,
             xidx_v, tidx_v, lidx_v, a_v, tonew_v, langw_v, mask_v, xlen_v, sem):
    wid = lax.axis_index("s") * NC + lax.axis_index("c")
    wbase = wid * ROWS_PER_W

    pltpu.sync_copy(tonew_hbm, tonew_v)
    pltpu.sync_copy(langw_hbm, langw_v)
    pltpu.sync_copy(xlen_hbm, xlen_v)

    iota = lax.iota(jnp.int32, LANES)

    for i in range(B_PER_W):
        b = wid * B_PER_W + i
        lenvec = plsc.load_gather(xlen_v, [jnp.full((LANES,), b, jnp.int32)])
        for j in range(L // LANES):
            col = iota + (LANES * j)
            m = jnp.where(col < lenvec, jnp.float32(1.0), jnp.float32(0.0))
            mask_v[pl.ds(i * L + LANES * j, LANES)] = m
    pltpu.sync_copy(mask_v, out_m.at[pl.ds(wid * (B_PER_W * L), B_PER_W * L)])

    for c in range(NCHUNK):
        base = wbase + c * CHUNK
        pltpu.sync_copy(x_hbm.at[pl.ds(base, CHUNK)], xidx_v)
        pltpu.sync_copy(tone_hbm.at[pl.ds(base, CHUNK)], tidx_v)
        pltpu.sync_copy(lang_hbm.at[pl.ds(base, CHUNK)], lidx_v)
        pltpu.async_copy(emb_hbm.at[xidx_v], a_v, sem).wait()

        def row_body(r, carry):
            rfull = jnp.full((LANES,), r, jnp.int32)
            tv = plsc.load_gather(tidx_v, [rfull])
            lv = plsc.load_gather(lidx_v, [rfull])
            for j in range(NSLICE):
                col = iota + (LANES * j)
                ts = plsc.load_gather(tonew_v, [tv, col])
                ls = plsc.load_gather(langw_v, [lv, col])
                va = plsc.load_gather(a_v, [rfull, col])
                plsc.store_scatter(a_v, [rfull, col], va + ts + ls)
            return carry

        lax.fori_loop(0, CHUNK, row_body, 0)
        pltpu.sync_copy(a_v, out_h.at[pl.ds(base, CHUNK)])


@jax.jit
def _sc_call(x_f, t_f, l_f, xl, emb_w, tone_w, lang_w):
    mesh = plsc.VectorSubcoreMesh(core_axis_name="c", subcore_axis_name="s")
    return pl.kernel(
        _sc_body,
        out_type=(
            jax.ShapeDtypeStruct((N, HIDDEN), jnp.float32),
            jax.ShapeDtypeStruct((B * L,), jnp.float32),
        ),
        mesh=mesh,
        scratch_types=[
            pltpu.VMEM((CHUNK,), jnp.int32),
            pltpu.VMEM((CHUNK,), jnp.int32),
            pltpu.VMEM((CHUNK,), jnp.int32),
            pltpu.VMEM((CHUNK, HIDDEN), jnp.float32),
            pltpu.VMEM((NUM_TONES, HIDDEN), jnp.float32),
            pltpu.VMEM((NUM_LANGUAGES, HIDDEN), jnp.float32),
            pltpu.VMEM((B_PER_W * L,), jnp.float32),
            pltpu.VMEM((B,), jnp.int32),
            pltpu.SemaphoreType.DMA,
        ],
        compiler_params=pltpu.CompilerParams(
            needs_layout_passes=False, use_tc_tiling_on_sc=False),
    )(x_f, t_f, l_f, xl, emb_w, tone_w, lang_w)


def kernel(x, x_lengths, tone, language, ja_bert, noise_scale, emb_w, tone_w, lang_w):
    x_f = x.reshape(-1).astype(jnp.int32)
    t_f = tone.reshape(-1).astype(jnp.int32)
    l_f = language.reshape(-1).astype(jnp.int32)
    xl = x_lengths.astype(jnp.int32)
    h_flat, mask = _sc_call(x_f, t_f, l_f, xl, emb_w, tone_w, lang_w)
    return h_flat.reshape(B, L, HIDDEN), mask.reshape(B, 1, L)
